# Initial kernel scaffold; baseline (speedup 1.0000x reference)
import jax
import jax.numpy as jnp
from jax.experimental import pallas as pl


def kernel(tokens, token_embedding, positional_embedding):
    raise NotImplementedError("write your pallas kernel here")


# speedup vs baseline: 2.2196x; 2.2196x over previous
import jax
import jax.numpy as jnp
from jax import lax
from jax.experimental import pallas as pl
from jax.experimental.pallas import tpu as pltpu
from jax.experimental.pallas import tpu_sc as plsc

VOCAB = 100000
EMBED = 64
NTOK = 200
BATCH = 4096

NC = 2
NS = 16
NW = NC * NS
SEQ_PER_W = BATCH // NW
HALF = NTOK // 2
NCHUNK = SEQ_PER_W * 2
LANES = 16


def _body(tokens_hbm, table_hbm, pos_hbm, out_hbm, idx_v, rows_v, pos_v, sem):
    wid = lax.axis_index("s") * NC + lax.axis_index("c")
    pltpu.sync_copy(tokens_hbm.at[wid], idx_v)
    pltpu.sync_copy(pos_hbm, pos_v)

    def seq(s, carry):
        cp0 = pltpu.async_copy(table_hbm.at[idx_v.at[2 * s]], rows_v.at[0], sem)
        cp1 = pltpu.async_copy(table_hbm.at[idx_v.at[2 * s + 1]], rows_v.at[1], sem)
        cp0.wait()
        cp1.wait()

        def radd(r, c2):
            for h in range(2):
                for c in range(EMBED // LANES):
                    ds = pl.ds(c * LANES, LANES)
                    rows_v[h, r, ds] = rows_v[h, r, ds] + pos_v[h, r, ds]
            return c2

        lax.fori_loop(0, HALF, radd, 0, unroll=2)
        pltpu.sync_copy(rows_v, out_hbm.at[wid * SEQ_PER_W + s])
        return carry

    lax.fori_loop(0, SEQ_PER_W, seq, 0)


def kernel(tokens, token_embedding, positional_embedding):
    tokens_r = tokens.reshape(NW, NCHUNK, HALF).astype(jnp.int32)
    pos_r = positional_embedding.reshape(2, HALF, EMBED)
    grid_kernel = pl.kernel(
        _body,
        out_type=jax.ShapeDtypeStruct((BATCH, 2, HALF, EMBED), jnp.float32),
        mesh=plsc.VectorSubcoreMesh(core_axis_name="c", subcore_axis_name="s"),
        compiler_params=pltpu.CompilerParams(use_tc_tiling_on_sc=False),
        scratch_types=[
            pltpu.VMEM((NCHUNK, HALF), jnp.int32),
            pltpu.VMEM((2, HALF, EMBED), jnp.float32),
            pltpu.VMEM((2, HALF, EMBED), jnp.float32),
            pltpu.SemaphoreType.DMA,
        ],
    )
    out = grid_kernel(tokens_r, token_embedding, pos_r)
    return out.reshape(BATCH, NTOK, EMBED)


# speedup vs baseline: 3.1115x; 1.4018x over previous
import jax
import jax.numpy as jnp
from jax import lax
from jax.experimental import pallas as pl
from jax.experimental.pallas import tpu as pltpu
from jax.experimental.pallas import tpu_sc as plsc

VOCAB = 100000
EMBED = 64
NTOK = 200
BATCH = 4096

NC = 2
NS = 16
NW = NC * NS
SEQ_PER_W = BATCH // NW
HALF = NTOK // 2
NCHUNK = SEQ_PER_W * 2
LANES = 16
NBUF = 3


def _body(tokens_hbm, table_hbm, pos_hbm, out_hbm, idx_v, rows_v,
          pos_v, g0, g1, g2, w0, w1, w2):
    sem_g = (g0, g1, g2)
    sem_w = (w0, w1, w2)
    wid = lax.axis_index("s") * NC + lax.axis_index("c")
    pltpu.sync_copy(tokens_hbm.at[wid], idx_v)
    pltpu.sync_copy(pos_hbm, pos_v)
    obase = wid * SEQ_PER_W

    def start_gather(s, b):
        pltpu.async_copy(table_hbm.at[idx_v.at[2 * s]], rows_v.at[b, 0], sem_g[b])
        pltpu.async_copy(table_hbm.at[idx_v.at[2 * s + 1]], rows_v.at[b, 1], sem_g[b])

    def wait_gather(s, b):
        pltpu.make_async_copy(table_hbm.at[idx_v.at[2 * s]],
                              rows_v.at[b, 0], sem_g[b]).wait()
        pltpu.make_async_copy(table_hbm.at[idx_v.at[2 * s + 1]],
                              rows_v.at[b, 1], sem_g[b]).wait()

    def start_write(s, b):
        pltpu.async_copy(rows_v.at[b], out_hbm.at[obase + s], sem_w[b])

    def wait_write(s, b):
        pltpu.make_async_copy(rows_v.at[b], out_hbm.at[obase + s],
                              sem_w[b]).wait()

    def add_pos(b):
        def radd(r, c2):
            for h in range(2):
                for c in range(EMBED // LANES):
                    ds = pl.ds(c * LANES, LANES)
                    rows_v[b, h, r, ds] = rows_v[b, h, r, ds] + pos_v[h, r, ds]
            return c2

        lax.fori_loop(0, HALF, radd, 0, unroll=4)

    def seq_body(s, b, prefetch, reclaim):
        pb = (b + 2) % NBUF
        if prefetch:
            if reclaim:
                wait_write(s - 1, pb)
            start_gather(s + 2, pb)
        wait_gather(s, b)
        add_pos(b)
        start_write(s, b)

    start_gather(0, 0)
    start_gather(1, 1)
    seq_body(0, 0, prefetch=True, reclaim=False)

    def outer(k, carry):
        s0 = 1 + 3 * k
        for j, b in enumerate((1, 2, 0)):
            seq_body(s0 + j, b, prefetch=True, reclaim=True)
        return carry

    lax.fori_loop(0, 41, outer, 0)
    seq_body(124, 1, prefetch=True, reclaim=True)
    seq_body(125, 2, prefetch=True, reclaim=True)
    seq_body(126, 0, prefetch=False, reclaim=False)
    seq_body(127, 1, prefetch=False, reclaim=False)
    wait_write(125, 2)
    wait_write(126, 0)
    wait_write(127, 1)


def kernel(tokens, token_embedding, positional_embedding):
    tokens_r = tokens.reshape(NW, NCHUNK, HALF).astype(jnp.int32)
    pos_r = positional_embedding.reshape(2, HALF, EMBED)
    grid_kernel = pl.kernel(
        _body,
        out_type=jax.ShapeDtypeStruct((BATCH, 2, HALF, EMBED), jnp.float32),
        mesh=plsc.VectorSubcoreMesh(core_axis_name="c", subcore_axis_name="s"),
        compiler_params=pltpu.CompilerParams(use_tc_tiling_on_sc=False),
        scratch_types=[
            pltpu.VMEM((NCHUNK, HALF), jnp.int32),
            pltpu.VMEM((NBUF, 2, HALF, EMBED), jnp.float32),
            pltpu.VMEM((2, HALF, EMBED), jnp.float32),
            pltpu.SemaphoreType.DMA,
            pltpu.SemaphoreType.DMA,
            pltpu.SemaphoreType.DMA,
            pltpu.SemaphoreType.DMA,
            pltpu.SemaphoreType.DMA,
            pltpu.SemaphoreType.DMA,
        ],
    )
    out = grid_kernel(tokens_r, token_embedding, pos_r)
    return out.reshape(BATCH, NTOK, EMBED)
